# baseline (device time: 214624 ns/iter reference)
import jax
import jax.numpy as jnp
from jax import lax
from jax.experimental import pallas as pl
from jax.experimental.pallas import tpu as pltpu

N_DEV = 32
S = 4
HR = 16
HL = 15

_TABLES = None


def _ring_tables():
    global _TABLES
    if _TABLES is not None:
        return _TABLES
    by_coords = {}
    for dv in jax.devices():
        coc = getattr(dv, "core_on_chip", None)
        if coc is not None and coc != 1:
            continue
        by_coords[tuple(getattr(dv, "coords", (dv.id,)))] = dv
    all_coords = sorted(by_coords)
    logical = []
    zs = sorted({c[2] for c in all_coords})
    for z in zs:
        plane = sorted(c for c in all_coords if c[2] == z)
        ys = sorted({c[1] for c in plane})
        for yi, y in enumerate(ys):
            row = sorted((c for c in plane if c[1] == y),
                         reverse=bool(yi % 2))
            logical.extend(row)
    logical = logical[:N_DEV]
    lidx = {c: i for i, c in enumerate(logical)}
    xs = sorted({c[0] for c in logical})
    ys = sorted({c[1] for c in logical})
    zs = sorted({c[2] for c in logical})
    if (len(xs), len(ys), len(zs)) == (2, 4, 4) and len(logical) == N_DEV:
        B = [(0, 0), (0, 1), (0, 2), (0, 3), (1, 3), (1, 2), (1, 1),
             (2, 1), (2, 2), (2, 3), (3, 3), (3, 2), (3, 1), (3, 0),
             (2, 0), (1, 0)]
        ring = ([(xs[0], ys[y], zs[z]) for (y, z) in B]
                + [(xs[1], ys[y], zs[z]) for (y, z) in reversed(B)])
        perm = [lidx[c] for c in ring]
    else:
        perm = list(range(len(logical)))
    inv = [0] * len(perm)
    for p, l in enumerate(perm):
        inv[l] = p
    _TABLES = (perm, inv)
    return _TABLES


def _gelu(y):
    c = 0.7978845608028654
    return 0.5 * y * (1.0 + jnp.tanh(c * (y + 0.044715 * y * y * y)))


def kernel(x, w_mat):
    m, k_loc = x.shape
    _, n = w_mat.shape
    chunk = m // N_DEV
    nsub = n // S

    perm_l, inv_l = _ring_tables()
    perm_arr = jnp.asarray(perm_l, jnp.int32)
    inv_arr = jnp.asarray(inv_l, jnp.int32)

    def body(x_ref, w_ref, perm_ref, inv_ref, out_ref, part,
             rs0, rs1,
             rss0, rsr0, rss1, rsr1,
             ags0, agr0, ags1, agr1):
        d = lax.axis_index("i")
        p = inv_ref[d]

        RS = (rs0, rs1)
        RSS = (rss0, rss1)
        RSR = (rsr0, rsr1)
        AGS = (ags0, ags1)
        AGR = (agr0, agr1)
        SIG = (1, -1)
        NH = (HR, HL)
        NBR = tuple(
            perm_ref[lax.rem(p + (N_DEV + sg), N_DEV)] for sg in SIG)

        def pmod(off):
            return lax.rem(p + (off % N_DEV), N_DEV)

        def rows(c):
            return pl.ds(c * chunk, chunk)

        def scols(s):
            return pl.ds(s * nsub, nsub)

        def piece(off):
            r = rows(pmod(off))
            part[r, :] = jnp.dot(
                x_ref[r, :], w_ref[:, :],
                preferred_element_type=jnp.float32,
            )

        def rs_send_chunk(di, h):
            return (HR - h) if di == 0 else (h - HL)

        def rs_recv_chunk(di, j):
            return (HR - 1 - j) if di == 0 else (j - HL + 1)

        def rs_rdma(di, h, s):
            if h == 0:
                src = part.at[rows(pmod(rs_send_chunk(di, 0))), scols(s)]
            else:
                src = RS[di].at[h - 1, s]
            return pltpu.make_async_remote_copy(
                src_ref=src,
                dst_ref=RS[di].at[h, s],
                send_sem=RSS[di].at[h, s],
                recv_sem=RSR[di].at[h, s],
                device_id=(NBR[di],),
                device_id_type=pl.DeviceIdType.MESH,
            )

        def ag_fwd_chunk(di, g):
            return -g if di == 0 else g

        def ag_rdma(di, g, s):
            r = rows(pmod(ag_fwd_chunk(di, g)))
            return pltpu.make_async_remote_copy(
                src_ref=out_ref.at[r, scols(s)],
                dst_ref=out_ref.at[r, scols(s)],
                send_sem=AGS[di].at[g, s],
                recv_sem=AGR[di].at[g, s],
                device_id=(NBR[di],),
                device_id_type=pl.DeviceIdType.MESH,
            )

        def ag_wait(di, g, s):
            r = rows(pmod(ag_fwd_chunk(di, g) - SIG[di]))
            return pltpu.make_async_remote_copy(
                src_ref=out_ref.at[r, scols(s)],
                dst_ref=out_ref.at[r, scols(s)],
                send_sem=AGS[di].at[g, s],
                recv_sem=AGR[di].at[g, s],
                device_id=(NBR[di],),
                device_id_type=pl.DeviceIdType.MESH,
            )

        for off in (HR, HR - 1, HR - 2, -HL, -HL + 1, -HL + 2):
            piece(off)

        barrier_sem = pltpu.get_barrier_semaphore()
        for di in (0, 1):
            pl.semaphore_signal(
                barrier_sem, inc=1,
                device_id=(NBR[di],),
                device_id_type=pl.DeviceIdType.MESH,
            )
        pl.semaphore_wait(barrier_sem, 2)

        for s in range(S):
            for di in (0, 1):
                rs_rdma(di, 0, s).start()

        for j in range(HR):
            for s in range(S):
                for di in (0, 1):
                    if j >= NH[di]:
                        continue
                    rs_rdma(di, j, s).wait_recv()
                    if j < NH[di] - 1:
                        RS[di][j, s] = (
                            RS[di][j, s]
                            + part[rows(pmod(rs_recv_chunk(di, j))),
                                   scols(s)])
                        rs_rdma(di, j + 1, s).start()
            if j <= HR - 3:
                piece(HR - 3 - j)
            if j <= HL - 4:
                piece(-(HL - 3) + j)

        for s in range(S):
            out_ref[rows(p), scols(s)] = (
                rs0[HR - 1, s] + rs1[HL - 1, s]
                + part[rows(p), scols(s)])
        out_ref[rows(p), :] = _gelu(out_ref[rows(p), :])

        for s in range(S):
            for di in (0, 1):
                ag_rdma(di, 0, s).start()
        for g in range(HR):
            for s in range(S):
                for di in (0, 1):
                    if g >= NH[di]:
                        continue
                    ag_wait(di, g, s).wait_recv()
                    if g < NH[di] - 1:
                        ag_rdma(di, g + 1, s).start()

        for h in range(HR):
            for s in range(S):
                for di in (0, 1):
                    if h < NH[di]:
                        rs_rdma(di, h, s).wait_send()
                        ag_rdma(di, h, s).wait_send()

    return pl.pallas_call(
        body,
        out_shape=jax.ShapeDtypeStruct((m, n), jnp.float32),
        in_specs=[
            pl.BlockSpec(memory_space=pltpu.VMEM),
            pl.BlockSpec(memory_space=pltpu.VMEM),
            pl.BlockSpec(memory_space=pltpu.SMEM),
            pl.BlockSpec(memory_space=pltpu.SMEM),
        ],
        out_specs=pl.BlockSpec(memory_space=pltpu.VMEM),
        scratch_shapes=[
            pltpu.VMEM((m, n), jnp.float32),
            pltpu.VMEM((HR, S, chunk, nsub), jnp.float32),
            pltpu.VMEM((HL, S, chunk, nsub), jnp.float32),
            pltpu.SemaphoreType.DMA((HR, S)),
            pltpu.SemaphoreType.DMA((HR, S)),
            pltpu.SemaphoreType.DMA((HL, S)),
            pltpu.SemaphoreType.DMA((HL, S)),
            pltpu.SemaphoreType.DMA((HR, S)),
            pltpu.SemaphoreType.DMA((HR, S)),
            pltpu.SemaphoreType.DMA((HL, S)),
            pltpu.SemaphoreType.DMA((HL, S)),
        ],
        compiler_params=pltpu.CompilerParams(
            vmem_limit_bytes=60 * 1024 * 1024,
            collective_id=0,
        ),
    )(x, w_mat, perm_arr, inv_arr)


# device time: 211843 ns/iter; 1.0131x vs baseline; 1.0131x over previous
import jax
import jax.numpy as jnp
from jax import lax
from jax.experimental import pallas as pl
from jax.experimental.pallas import tpu as pltpu

N_DEV = 32
S = 2
HR = 16
HL = 15

_TABLES = None


def _ring_tables():
    global _TABLES
    if _TABLES is not None:
        return _TABLES
    by_coords = {}
    for dv in jax.devices():
        coc = getattr(dv, "core_on_chip", None)
        if coc is not None and coc != 1:
            continue
        by_coords[tuple(getattr(dv, "coords", (dv.id,)))] = dv
    all_coords = sorted(by_coords)
    logical = []
    zs = sorted({c[2] for c in all_coords})
    for z in zs:
        plane = sorted(c for c in all_coords if c[2] == z)
        ys = sorted({c[1] for c in plane})
        for yi, y in enumerate(ys):
            row = sorted((c for c in plane if c[1] == y),
                         reverse=bool(yi % 2))
            logical.extend(row)
    logical = logical[:N_DEV]
    lidx = {c: i for i, c in enumerate(logical)}
    xs = sorted({c[0] for c in logical})
    ys = sorted({c[1] for c in logical})
    zs = sorted({c[2] for c in logical})
    if (len(xs), len(ys), len(zs)) == (2, 4, 4) and len(logical) == N_DEV:
        B = [(0, 0), (0, 1), (0, 2), (0, 3), (1, 3), (1, 2), (1, 1),
             (2, 1), (2, 2), (2, 3), (3, 3), (3, 2), (3, 1), (3, 0),
             (2, 0), (1, 0)]
        ring = ([(xs[0], ys[y], zs[z]) for (y, z) in B]
                + [(xs[1], ys[y], zs[z]) for (y, z) in reversed(B)])
        perm = [lidx[c] for c in ring]
    else:
        perm = list(range(len(logical)))
    inv = [0] * len(perm)
    for p, l in enumerate(perm):
        inv[l] = p
    _TABLES = (perm, inv)
    return _TABLES


def _gelu(y):
    c = 0.7978845608028654
    return 0.5 * y * (1.0 + jnp.tanh(c * (y + 0.044715 * y * y * y)))


def kernel(x, w_mat):
    m, k_loc = x.shape
    _, n = w_mat.shape
    chunk = m // N_DEV
    nsub = n // S

    perm_l, inv_l = _ring_tables()
    perm_arr = jnp.asarray(perm_l, jnp.int32)
    inv_arr = jnp.asarray(inv_l, jnp.int32)

    def body(x_ref, w_ref, perm_ref, inv_ref, out_ref, part,
             rs0, rs1,
             rss0, rsr0, rss1, rsr1,
             ags0, agr0, ags1, agr1):
        d = lax.axis_index("i")
        p = inv_ref[d]

        RS = (rs0, rs1)
        RSS = (rss0, rss1)
        RSR = (rsr0, rsr1)
        AGS = (ags0, ags1)
        AGR = (agr0, agr1)
        SIG = (1, -1)
        NH = (HR, HL)
        NBR = tuple(
            perm_ref[lax.rem(p + (N_DEV + sg), N_DEV)] for sg in SIG)

        def pmod(off):
            return lax.rem(p + (off % N_DEV), N_DEV)

        def rows(c):
            return pl.ds(c * chunk, chunk)

        def scols(s):
            return pl.ds(s * nsub, nsub)

        def piece(off):
            r = rows(pmod(off))
            part[r, :] = jnp.dot(
                x_ref[r, :], w_ref[:, :],
                preferred_element_type=jnp.float32,
            )

        def rs_send_chunk(di, h):
            return (HR - h) if di == 0 else (h - HL)

        def rs_recv_chunk(di, j):
            return (HR - 1 - j) if di == 0 else (j - HL + 1)

        def rs_rdma(di, h, s):
            if h == 0:
                src = part.at[rows(pmod(rs_send_chunk(di, 0))), scols(s)]
            else:
                src = RS[di].at[h - 1, s]
            return pltpu.make_async_remote_copy(
                src_ref=src,
                dst_ref=RS[di].at[h, s],
                send_sem=RSS[di].at[h, s],
                recv_sem=RSR[di].at[h, s],
                device_id=(NBR[di],),
                device_id_type=pl.DeviceIdType.MESH,
            )

        def ag_fwd_chunk(di, g):
            return -g if di == 0 else g

        def ag_rdma(di, g, s):
            r = rows(pmod(ag_fwd_chunk(di, g)))
            return pltpu.make_async_remote_copy(
                src_ref=out_ref.at[r, scols(s)],
                dst_ref=out_ref.at[r, scols(s)],
                send_sem=AGS[di].at[g, s],
                recv_sem=AGR[di].at[g, s],
                device_id=(NBR[di],),
                device_id_type=pl.DeviceIdType.MESH,
            )

        def ag_wait(di, g, s):
            r = rows(pmod(ag_fwd_chunk(di, g) - SIG[di]))
            return pltpu.make_async_remote_copy(
                src_ref=out_ref.at[r, scols(s)],
                dst_ref=out_ref.at[r, scols(s)],
                send_sem=AGS[di].at[g, s],
                recv_sem=AGR[di].at[g, s],
                device_id=(NBR[di],),
                device_id_type=pl.DeviceIdType.MESH,
            )

        piece(HR)
        piece(-HL)

        barrier_sem = pltpu.get_barrier_semaphore()
        for di in (0, 1):
            pl.semaphore_signal(
                barrier_sem, inc=1,
                device_id=(NBR[di],),
                device_id_type=pl.DeviceIdType.MESH,
            )
        pl.semaphore_wait(barrier_sem, 2)

        for s in range(S):
            for di in (0, 1):
                rs_rdma(di, 0, s).start()

        for off in (HR - 1, -HL + 1, HR - 2, -HL + 2):
            piece(off)

        for j in range(HR):
            for s in range(S):
                for di in (0, 1):
                    if j >= NH[di]:
                        continue
                    rs_rdma(di, j, s).wait_recv()
                    if j < NH[di] - 1:
                        RS[di][j, s] = (
                            RS[di][j, s]
                            + part[rows(pmod(rs_recv_chunk(di, j))),
                                   scols(s)])
                        rs_rdma(di, j + 1, s).start()
            if j <= HR - 3:
                piece(HR - 3 - j)
            if j <= HL - 4:
                piece(-(HL - 3) + j)

        for s in range(S):
            out_ref[rows(p), scols(s)] = _gelu(
                rs0[HR - 1, s] + rs1[HL - 1, s]
                + part[rows(p), scols(s)])
            for di in (0, 1):
                ag_rdma(di, 0, s).start()
        for g in range(HR):
            for s in range(S):
                for di in (0, 1):
                    if g >= NH[di]:
                        continue
                    ag_wait(di, g, s).wait_recv()
                    if g < NH[di] - 1:
                        ag_rdma(di, g + 1, s).start()

        for h in range(HR):
            for s in range(S):
                for di in (0, 1):
                    if h < NH[di]:
                        rs_rdma(di, h, s).wait_send()
                        ag_rdma(di, h, s).wait_send()

    return pl.pallas_call(
        body,
        out_shape=jax.ShapeDtypeStruct((m, n), jnp.float32),
        in_specs=[
            pl.BlockSpec(memory_space=pltpu.VMEM),
            pl.BlockSpec(memory_space=pltpu.VMEM),
            pl.BlockSpec(memory_space=pltpu.SMEM),
            pl.BlockSpec(memory_space=pltpu.SMEM),
        ],
        out_specs=pl.BlockSpec(memory_space=pltpu.VMEM),
        scratch_shapes=[
            pltpu.VMEM((m, n), jnp.float32),
            pltpu.VMEM((HR, S, chunk, nsub), jnp.float32),
            pltpu.VMEM((HL, S, chunk, nsub), jnp.float32),
            pltpu.SemaphoreType.DMA((HR, S)),
            pltpu.SemaphoreType.DMA((HR, S)),
            pltpu.SemaphoreType.DMA((HL, S)),
            pltpu.SemaphoreType.DMA((HL, S)),
            pltpu.SemaphoreType.DMA((HR, S)),
            pltpu.SemaphoreType.DMA((HR, S)),
            pltpu.SemaphoreType.DMA((HL, S)),
            pltpu.SemaphoreType.DMA((HL, S)),
        ],
        compiler_params=pltpu.CompilerParams(
            vmem_limit_bytes=60 * 1024 * 1024,
            collective_id=0,
        ),
    )(x, w_mat, perm_arr, inv_arr)


# device time: 209967 ns/iter; 1.0222x vs baseline; 1.0089x over previous
import jax
import jax.numpy as jnp
from jax import lax
from jax.experimental import pallas as pl
from jax.experimental.pallas import tpu as pltpu

N_DEV = 32
AH = ((16, 15), (15, 16))

_TABLES = None


def _ring_tables():
    global _TABLES
    if _TABLES is not None:
        return _TABLES
    by_coords = {}
    for dv in jax.devices():
        coc = getattr(dv, "core_on_chip", None)
        if coc is not None and coc != 1:
            continue
        by_coords[tuple(getattr(dv, "coords", (dv.id,)))] = dv
    all_coords = sorted(by_coords)
    logical = []
    zs = sorted({c[2] for c in all_coords})
    for z in zs:
        plane = sorted(c for c in all_coords if c[2] == z)
        ys = sorted({c[1] for c in plane})
        for yi, y in enumerate(ys):
            row = sorted((c for c in plane if c[1] == y),
                         reverse=bool(yi % 2))
            logical.extend(row)
    logical = logical[:N_DEV]
    lidx = {c: i for i, c in enumerate(logical)}
    xs = sorted({c[0] for c in logical})
    ys = sorted({c[1] for c in logical})
    zs = sorted({c[2] for c in logical})
    if (len(xs), len(ys), len(zs)) == (2, 4, 4) and len(logical) == N_DEV:
        B = [(0, 0), (0, 1), (0, 2), (0, 3), (1, 3), (1, 2), (1, 1),
             (2, 1), (2, 2), (2, 3), (3, 3), (3, 2), (3, 1), (3, 0),
             (2, 0), (1, 0)]
        ring = ([(xs[0], ys[y], zs[z]) for (y, z) in B]
                + [(xs[1], ys[y], zs[z]) for (y, z) in reversed(B)])
        perm = [lidx[c] for c in ring]
    else:
        perm = list(range(len(logical)))
    inv = [0] * len(perm)
    for p, l in enumerate(perm):
        inv[l] = p
    _TABLES = (perm, inv)
    return _TABLES


def _gelu(y):
    c = 0.7978845608028654
    return 0.5 * y * (1.0 + jnp.tanh(c * (y + 0.044715 * y * y * y)))


def kernel(x, w_mat):
    m, k_loc = x.shape
    _, n = w_mat.shape
    chunk = m // N_DEV
    half = n // 2

    perm_l, inv_l = _ring_tables()
    perm_arr = jnp.asarray(perm_l, jnp.int32)
    inv_arr = jnp.asarray(inv_l, jnp.int32)

    def body(x_ref, w_ref, perm_ref, inv_ref, out_ref, part,
             rsA0, rsA1, rsB0, rsB1,
             rssA0, rsrA0, rssA1, rsrA1,
             rssB0, rsrB0, rssB1, rsrB1,
             agsA0, agrA0, agsA1, agrA1,
             agsB0, agrB0, agsB1, agrB1):
        d = lax.axis_index("i")
        p = inv_ref[d]

        RSB = {(0, 0): rsA0, (0, 1): rsA1, (1, 0): rsB0, (1, 1): rsB1}
        RSS = {(0, 0): rssA0, (0, 1): rssA1, (1, 0): rssB0, (1, 1): rssB1}
        RSR = {(0, 0): rsrA0, (0, 1): rsrA1, (1, 0): rsrB0, (1, 1): rsrB1}
        AGS = {(0, 0): agsA0, (0, 1): agsA1, (1, 0): agsB0, (1, 1): agsB1}
        AGR = {(0, 0): agrA0, (0, 1): agrA1, (1, 0): agrB0, (1, 1): agrB1}
        SIG = (1, -1)
        NBR = tuple(
            perm_ref[lax.rem(p + (N_DEV + sg), N_DEV)] for sg in SIG)

        def pmod(off):
            return lax.rem(p + (off % N_DEV), N_DEV)

        def rows(c):
            return pl.ds(c * chunk, chunk)

        def scols(st):
            return pl.ds(st * half, half)

        def piece(off):
            r = rows(pmod(off))
            part[r, :] = jnp.dot(
                x_ref[r, :], w_ref[:, :],
                preferred_element_type=jnp.float32,
            )

        def rs_send_chunk(st, di, h):
            a = AH[st][di]
            return SIG[di] * (a - h)

        def rs_recv_chunk(st, di, j):
            a = AH[st][di]
            return SIG[di] * (a - 1 - j)

        def rs_rdma(st, di, h):
            if h == 0:
                src = part.at[rows(pmod(rs_send_chunk(st, di, 0))),
                              scols(st)]
            else:
                src = RSB[st, di].at[h - 1]
            return pltpu.make_async_remote_copy(
                src_ref=src,
                dst_ref=RSB[st, di].at[h],
                send_sem=RSS[st, di].at[h],
                recv_sem=RSR[st, di].at[h],
                device_id=(NBR[di],),
                device_id_type=pl.DeviceIdType.MESH,
            )

        def ag_rdma(st, di, g):
            r = rows(pmod(-SIG[di] * g))
            return pltpu.make_async_remote_copy(
                src_ref=out_ref.at[r, scols(st)],
                dst_ref=out_ref.at[r, scols(st)],
                send_sem=AGS[st, di].at[g],
                recv_sem=AGR[st, di].at[g],
                device_id=(NBR[di],),
                device_id_type=pl.DeviceIdType.MESH,
            )

        def ag_wait(st, di, g):
            r = rows(pmod(-SIG[di] * (g + 1)))
            return pltpu.make_async_remote_copy(
                src_ref=out_ref.at[r, scols(st)],
                dst_ref=out_ref.at[r, scols(st)],
                send_sem=AGS[st, di].at[g],
                recv_sem=AGR[st, di].at[g],
                device_id=(NBR[di],),
                device_id_type=pl.DeviceIdType.MESH,
            )

        for off in (16, 15, -15):
            piece(off)

        barrier_sem = pltpu.get_barrier_semaphore()
        for di in (0, 1):
            pl.semaphore_signal(
                barrier_sem, inc=1,
                device_id=(NBR[di],),
                device_id_type=pl.DeviceIdType.MESH,
            )
        pl.semaphore_wait(barrier_sem, 2)

        for st in (0, 1):
            for di in (0, 1):
                rs_rdma(st, di, 0).start()

        for off in (14, -14, 13, -13):
            piece(off)

        for j in range(16):
            for st in (0, 1):
                for di in (0, 1):
                    a = AH[st][di]
                    if j >= a:
                        continue
                    rs_rdma(st, di, j).wait_recv()
                    if j < a - 1:
                        RSB[st, di][j] = (
                            RSB[st, di][j]
                            + part[rows(pmod(rs_recv_chunk(st, di, j))),
                                   scols(st)])
                        rs_rdma(st, di, j + 1).start()
            if j <= 11:
                piece(12 - j)
                piece(-(12 - j))
            if j == 12:
                piece(0)

        for st in (0, 1):
            out_ref[rows(p), scols(st)] = _gelu(
                RSB[st, 0][AH[st][0] - 1]
                + RSB[st, 1][AH[st][1] - 1]
                + part[rows(p), scols(st)])
            for di in (0, 1):
                ag_rdma(st, di, 0).start()

        for g in range(16):
            for st in (0, 1):
                for di in (0, 1):
                    a = AH[st][di]
                    if g >= a:
                        continue
                    ag_wait(st, di, g).wait_recv()
                    if g < a - 1:
                        ag_rdma(st, di, g + 1).start()

        for h in range(16):
            for st in (0, 1):
                for di in (0, 1):
                    if h < AH[st][di]:
                        rs_rdma(st, di, h).wait_send()
                        ag_rdma(st, di, h).wait_send()

    return pl.pallas_call(
        body,
        out_shape=jax.ShapeDtypeStruct((m, n), jnp.float32),
        in_specs=[
            pl.BlockSpec(memory_space=pltpu.VMEM),
            pl.BlockSpec(memory_space=pltpu.VMEM),
            pl.BlockSpec(memory_space=pltpu.SMEM),
            pl.BlockSpec(memory_space=pltpu.SMEM),
        ],
        out_specs=pl.BlockSpec(memory_space=pltpu.VMEM),
        scratch_shapes=[
            pltpu.VMEM((m, n), jnp.float32),
            pltpu.VMEM((16, chunk, half), jnp.float32),
            pltpu.VMEM((15, chunk, half), jnp.float32),
            pltpu.VMEM((15, chunk, half), jnp.float32),
            pltpu.VMEM((16, chunk, half), jnp.float32),
            pltpu.SemaphoreType.DMA((16,)),
            pltpu.SemaphoreType.DMA((16,)),
            pltpu.SemaphoreType.DMA((15,)),
            pltpu.SemaphoreType.DMA((15,)),
            pltpu.SemaphoreType.DMA((15,)),
            pltpu.SemaphoreType.DMA((15,)),
            pltpu.SemaphoreType.DMA((16,)),
            pltpu.SemaphoreType.DMA((16,)),
            pltpu.SemaphoreType.DMA((16,)),
            pltpu.SemaphoreType.DMA((16,)),
            pltpu.SemaphoreType.DMA((15,)),
            pltpu.SemaphoreType.DMA((15,)),
            pltpu.SemaphoreType.DMA((15,)),
            pltpu.SemaphoreType.DMA((15,)),
            pltpu.SemaphoreType.DMA((16,)),
            pltpu.SemaphoreType.DMA((16,)),
        ],
        compiler_params=pltpu.CompilerParams(
            vmem_limit_bytes=60 * 1024 * 1024,
            collective_id=0,
        ),
    )(x, w_mat, perm_arr, inv_arr)
